# Initial kernel scaffold; baseline (speedup 1.0000x reference)
import jax
import jax.numpy as jnp
from jax.experimental import pallas as pl


def kernel(node_emb, edge_index, W1, b1, W2, b2):
    raise NotImplementedError("write your pallas kernel here")


# speedup vs baseline: 2.4537x; 2.4537x over previous
import functools

import numpy as np
import jax
import jax.numpy as jnp
from jax import lax
from jax.experimental import pallas as pl
from jax.experimental.pallas import tpu as pltpu
from jax.experimental.pallas import tpu_sc as plsc

_LANES = 16

_LO = float(np.log(np.float32(0.01)) - np.log1p(np.float32(-0.01)))
_HI = float(np.log(np.float32(0.99)) - np.log1p(np.float32(-0.99)))


def _tc_tables_body(emb, w1a, w1b, b1r, p_out, q_out):
    x = emb[...]
    p_out[...] = jnp.dot(x, w1a[...], preferred_element_type=jnp.float32)
    q_out[...] = jnp.dot(x, w1b[...], preferred_element_type=jnp.float32) + b1r[...]


def _make_tables(node_emb, W1, b1):
    n, d = node_emb.shape
    h = W1.shape[1]
    w1a = W1[:d]
    w1b = W1[d:]
    b1r = b1.reshape(1, h)
    bm = 1000 if n % 1000 == 0 else n
    grid = n // bm
    return pl.pallas_call(
        _tc_tables_body,
        grid=(grid,),
        in_specs=[
            pl.BlockSpec((bm, d), lambda i: (i, 0)),
            pl.BlockSpec((d, h), lambda i: (0, 0)),
            pl.BlockSpec((d, h), lambda i: (0, 0)),
            pl.BlockSpec((1, h), lambda i: (0, 0)),
        ],
        out_specs=[
            pl.BlockSpec((bm, h), lambda i: (i, 0)),
            pl.BlockSpec((bm, h), lambda i: (i, 0)),
        ],
        out_shape=[
            jax.ShapeDtypeStruct((n, h), jnp.float32),
            jax.ShapeDtypeStruct((n, h), jnp.float32),
        ],
    )(node_emb, w1a, w1b, b1r)


@functools.cache
def _make_sc_kernel(e, h, nc, ns, chunk):
    nw = nc * ns
    per_w = e // nw
    nchunks = per_w // chunk
    ngroups = chunk // _LANES
    mesh = plsc.VectorSubcoreMesh(core_axis_name="c", subcore_axis_name="s")

    @functools.partial(
        pl.kernel,
        out_type=[
            jax.ShapeDtypeStruct((e,), jnp.float32),
            jax.ShapeDtypeStruct((e,), jnp.int32),
            jax.ShapeDtypeStruct((e,), jnp.int32),
        ],
        mesh=mesh,
        compiler_params=pltpu.CompilerParams(
            needs_layout_passes=False, use_tc_tiling_on_sc=False),
        scratch_types=[
            pltpu.VMEM((chunk,), jnp.int32),
            pltpu.VMEM((chunk,), jnp.int32),
            pltpu.VMEM((chunk, h), jnp.float32),
            pltpu.VMEM((chunk, h), jnp.float32),
            pltpu.VMEM((chunk,), jnp.float32),
            pltpu.VMEM((chunk,), jnp.float32),
            pltpu.VMEM((chunk,), jnp.float32),
            pltpu.VMEM((chunk,), jnp.int32),
            pltpu.VMEM((chunk,), jnp.int32),
            pltpu.VMEM((h * _LANES,), jnp.float32),
            pltpu.SemaphoreType.DMA,
            pltpu.SemaphoreType.DMA,
        ],
    )
    def sc_kernel(p_hbm, q_hbm, src_hbm, dst_hbm, gn_hbm, lg_hbm, w2_hbm,
                  w_hbm, fs_hbm, fd_hbm,
                  sidx, didx, prow, qrow, gnv, lgv, wout, fsout, fdout, w2v,
                  sem1, sem2):
        wid = lax.axis_index("s") * nc + lax.axis_index("c")
        base = wid * per_w
        pltpu.sync_copy(w2_hbm, w2v)

        def chunk_body(c, carry):
            off = base + c * chunk
            pltpu.sync_copy(src_hbm.at[pl.ds(off, chunk)], sidx)
            pltpu.sync_copy(dst_hbm.at[pl.ds(off, chunk)], didx)
            pltpu.sync_copy(gn_hbm.at[pl.ds(off, chunk)], gnv)
            pltpu.sync_copy(lg_hbm.at[pl.ds(off, chunk)], lgv)
            cp1 = pltpu.async_copy(p_hbm.at[sidx], prow, sem1)
            cp2 = pltpu.async_copy(q_hbm.at[didx], qrow, sem2)
            cp1.wait()
            cp2.wait()

            def group_body(g, gcarry):
                r0 = g * _LANES
                rows = r0 + lax.iota(jnp.int32, _LANES)
                acc = jnp.zeros((_LANES,), jnp.float32)
                for j in range(h):
                    cols = jnp.full((_LANES,), j, jnp.int32)
                    pc = plsc.load_gather(prow, [rows, cols])
                    qc = plsc.load_gather(qrow, [rows, cols])
                    hv = jnp.maximum(pc + qc, 0.0)
                    acc = acc + hv * w2v[pl.ds(j * _LANES, _LANES)]
                g16 = acc + gnv[pl.ds(r0, _LANES)]
                lp = jnp.minimum(jnp.maximum(g16, _LO), _HI)
                z = (lp + lgv[pl.ds(r0, _LANES)]) / 0.9
                wv = 1.0 / (1.0 + jnp.exp(-z))
                keep = wv != 0.0
                wout[pl.ds(r0, _LANES)] = wv
                fsout[pl.ds(r0, _LANES)] = jnp.where(keep, sidx[pl.ds(r0, _LANES)], -1)
                fdout[pl.ds(r0, _LANES)] = jnp.where(keep, didx[pl.ds(r0, _LANES)], -1)
                return gcarry

            lax.fori_loop(0, ngroups, group_body, 0)
            pltpu.sync_copy(wout, w_hbm.at[pl.ds(off, chunk)])
            pltpu.sync_copy(fsout, fs_hbm.at[pl.ds(off, chunk)])
            pltpu.sync_copy(fdout, fd_hbm.at[pl.ds(off, chunk)])
            return carry

        lax.fori_loop(0, nchunks, chunk_body, 0)

    return sc_kernel


def kernel(node_emb, edge_index, W1, b1, W2, b2):
    e = edge_index.shape[1]
    h = W1.shape[1]

    p_tab, q_tab = _make_tables(node_emb, W1, b1)
    src = edge_index[0]
    dst = edge_index[1]

    u = jax.random.uniform(jax.random.key(1), (e, 1), dtype=jnp.float32)[:, 0]
    bias = 0.0001
    eps = (bias - (1.0 - bias)) * u + (1.0 - bias)
    gn = jnp.log(eps) - jnp.log(1.0 - eps) + b2[0]
    u2 = jax.random.uniform(jax.random.key(2), (e,), minval=1e-7,
                            maxval=1.0 - 1e-7, dtype=jnp.float32)
    lg = jnp.log(u2) - jnp.log(1.0 - u2)
    w2rep = jnp.repeat(W2[:, 0], _LANES)

    try:
        info = plsc.get_sparse_core_info()
        nc, ns = info.num_cores, info.num_subcores
    except Exception:
        nc, ns = 2, 16

    sck = _make_sc_kernel(e, h, nc, ns, 80)
    w, fs, fd = sck(p_tab, q_tab, src, dst, gn, lg, w2rep)
    return w, fs, fd


# speedup vs baseline: 3.1554x; 1.2860x over previous
import functools

import numpy as np
import jax
import jax.numpy as jnp
from jax import lax
from jax.experimental import pallas as pl
from jax.experimental.pallas import tpu as pltpu
from jax.experimental.pallas import tpu_sc as plsc

_LANES = 16

_LO = float(np.log(np.float32(0.01)) - np.log1p(np.float32(-0.01)))
_HI = float(np.log(np.float32(0.99)) - np.log1p(np.float32(-0.99)))


def _tc_tables_body(emb, w1a, w1b, b1r, p_out, q_out):
    x = emb[...]
    p_out[...] = jnp.dot(x, w1a[...], preferred_element_type=jnp.float32)
    q_out[...] = jnp.dot(x, w1b[...], preferred_element_type=jnp.float32) + b1r[...]


def _make_tables(node_emb, W1, b1):
    n, d = node_emb.shape
    h = W1.shape[1]
    w1a = W1[:d]
    w1b = W1[d:]
    b1r = b1.reshape(1, h)
    bm = 1000 if n % 1000 == 0 else n
    grid = n // bm
    return pl.pallas_call(
        _tc_tables_body,
        grid=(grid,),
        in_specs=[
            pl.BlockSpec((bm, d), lambda i: (i, 0)),
            pl.BlockSpec((d, h), lambda i: (0, 0)),
            pl.BlockSpec((d, h), lambda i: (0, 0)),
            pl.BlockSpec((1, h), lambda i: (0, 0)),
        ],
        out_specs=[
            pl.BlockSpec((bm, h), lambda i: (i, 0)),
            pl.BlockSpec((bm, h), lambda i: (i, 0)),
        ],
        out_shape=[
            jax.ShapeDtypeStruct((n, h), jnp.float32),
            jax.ShapeDtypeStruct((n, h), jnp.float32),
        ],
    )(node_emb, w1a, w1b, b1r)


@functools.cache
def _make_sc_kernel(e, h, nc, ns, chunk):
    nw = nc * ns
    per_w = e // nw
    nchunks = per_w // chunk
    ngroups = chunk // _LANES
    npairs = (nchunks - 1) // 2
    ilen = 4 * chunk
    olen = 3 * chunk
    mesh = plsc.VectorSubcoreMesh(core_axis_name="c", subcore_axis_name="s")

    @functools.partial(
        pl.kernel,
        out_type=jax.ShapeDtypeStruct((3 * e,), jnp.int32),
        mesh=mesh,
        compiler_params=pltpu.CompilerParams(
            needs_layout_passes=False, use_tc_tiling_on_sc=False),
        scratch_types=[
            pltpu.VMEM((ilen,), jnp.int32),
            pltpu.VMEM((ilen,), jnp.int32),
            pltpu.VMEM((chunk, h), jnp.float32),
            pltpu.VMEM((chunk, h), jnp.float32),
            pltpu.VMEM((chunk, h), jnp.float32),
            pltpu.VMEM((chunk, h), jnp.float32),
            pltpu.VMEM((olen,), jnp.int32),
            pltpu.VMEM((olen,), jnp.int32),
            pltpu.VMEM((h * _LANES,), jnp.float32),
            pltpu.SemaphoreType.DMA,
            pltpu.SemaphoreType.DMA,
            pltpu.SemaphoreType.DMA,
            pltpu.SemaphoreType.DMA,
            pltpu.SemaphoreType.DMA,
            pltpu.SemaphoreType.DMA,
            pltpu.SemaphoreType.DMA,
            pltpu.SemaphoreType.DMA,
        ],
    )
    def sc_kernel(p_hbm, q_hbm, in_hbm, w2_hbm, out_hbm,
                  inb0, inb1, pb0, pb1, qb0, qb1, ob0, ob1, w2v,
                  si0, si1, sp0, sp1, sq0, sq1, so0, so1):
        wid = lax.axis_index("s") * nc + lax.axis_index("c")
        cbase = wid * nchunks
        pltpu.sync_copy(w2_hbm, w2v)
        w2list = [w2v[pl.ds(j * _LANES, _LANES)] for j in range(h)]
        lane_off = lax.iota(jnp.int32, _LANES) * h
        zeros16 = jnp.zeros((_LANES,), jnp.int32)

        inbs = (inb0, inb1)
        pbs = (pb0, pb1)
        qbs = (qb0, qb1)
        obs = (ob0, ob1)
        sis = (si0, si1)
        sps = (sp0, sp1)
        sqs = (sq0, sq1)
        sos = (so0, so1)

        def in_desc(g, par):
            return pltpu.make_async_copy(
                in_hbm.at[pl.ds(g * ilen, ilen)], inbs[par], sis[par])

        def gather_descs(g, par):
            inb = inbs[par]
            cp = pltpu.make_async_copy(
                p_hbm.at[inb.at[pl.ds(0, chunk)]], pbs[par], sps[par])
            cq = pltpu.make_async_copy(
                q_hbm.at[inb.at[pl.ds(chunk, chunk)]], qbs[par], sqs[par])
            return cp, cq

        def out_desc(g, par):
            return pltpu.make_async_copy(
                obs[par], out_hbm.at[pl.ds(g * olen, olen)], sos[par])

        def compute(par):
            inb, pb, qb, ob = inbs[par], pbs[par], qbs[par], obs[par]

            def group(gi, carry):
                r0 = gi * _LANES
                idx = lane_off + r0 * h
                acc = jnp.zeros((_LANES,), jnp.float32)
                for j in range(h):
                    pc = plsc.load_gather(pb, [zeros16, idx])
                    qc = plsc.load_gather(qb, [zeros16, idx])
                    hv = jnp.maximum(pc + qc, 0.0)
                    acc = acc + hv * w2list[j]
                    if j + 1 < h:
                        idx = idx + 1
                gn16 = plsc.bitcast(inb[pl.ds(2 * chunk + r0, _LANES)], jnp.float32)
                lg16 = plsc.bitcast(inb[pl.ds(3 * chunk + r0, _LANES)], jnp.float32)
                g16 = acc + gn16
                lp = jnp.minimum(jnp.maximum(g16, _LO), _HI)
                z = (lp + lg16) / 0.9
                wv = 1.0 / (1.0 + jnp.exp(-z))
                keep = wv != 0.0
                s16 = inb[pl.ds(r0, _LANES)]
                d16 = inb[pl.ds(chunk + r0, _LANES)]
                ob[pl.ds(r0, _LANES)] = plsc.bitcast(wv, jnp.int32)
                ob[pl.ds(chunk + r0, _LANES)] = jnp.where(keep, s16, -1)
                ob[pl.ds(2 * chunk + r0, _LANES)] = jnp.where(keep, d16, -1)
                return carry

            lax.fori_loop(0, ngroups, group, 0)

        def sub(c, par, do_next_gather, do_in_guard, store_wait_dynamic):
            g = cbase + c
            gp, gq = gather_descs(g, par)
            gp.wait()
            gq.wait()
            if do_next_gather:
                in_desc(g + 1, par ^ 1).wait()
                np_, nq = gather_descs(g + 1, par ^ 1)
                np_.start()
                nq.start()
            if store_wait_dynamic:
                @pl.when(c >= 2)
                def _():
                    out_desc(g - 2, par).wait()
            else:
                out_desc(g - 2, par).wait()
            compute(par)
            out_desc(g, par).start()
            if do_in_guard == "always":
                in_desc(g + 2, par).start()
            elif do_in_guard == "guard":
                @pl.when(c + 2 <= nchunks - 1)
                def _():
                    in_desc(g + 2, par).start()

        in_desc(cbase, 0).start()
        in_desc(cbase + 1, 1).start()
        in_desc(cbase, 0).wait()
        gp0, gq0 = gather_descs(cbase, 0)
        gp0.start()
        gq0.start()

        def pair(i, carry):
            c0 = 2 * i
            sub(c0, 0, True, "always", True)
            sub(c0 + 1, 1, True, "guard", True)
            return carry

        lax.fori_loop(0, npairs, pair, 0)
        sub(nchunks - 1, 0, False, "none", False)
        out_desc(cbase + nchunks - 2, 1).wait()
        out_desc(cbase + nchunks - 1, 0).wait()

    return sc_kernel


def kernel(node_emb, edge_index, W1, b1, W2, b2):
    e = edge_index.shape[1]
    h = W1.shape[1]
    chunk = 80

    p_tab, q_tab = _make_tables(node_emb, W1, b1)
    src = edge_index[0]
    dst = edge_index[1]

    u = jax.random.uniform(jax.random.key(1), (e, 1), dtype=jnp.float32)[:, 0]
    bias = 0.0001
    eps = (bias - (1.0 - bias)) * u + (1.0 - bias)
    gn = jnp.log(eps) - jnp.log(1.0 - eps) + b2[0]
    u2 = jax.random.uniform(jax.random.key(2), (e,), minval=1e-7,
                            maxval=1.0 - 1e-7, dtype=jnp.float32)
    lg = jnp.log(u2) - jnp.log(1.0 - u2)
    w2rep = jnp.repeat(W2[:, 0], _LANES)

    gnb = lax.bitcast_convert_type(gn, jnp.int32)
    lgb = lax.bitcast_convert_type(lg, jnp.int32)
    inpk = jnp.stack(
        [src.reshape(-1, chunk), dst.reshape(-1, chunk),
         gnb.reshape(-1, chunk), lgb.reshape(-1, chunk)], axis=1).reshape(-1)

    try:
        info = plsc.get_sparse_core_info()
        nc, ns = info.num_cores, info.num_subcores
    except Exception:
        nc, ns = 2, 16

    sck = _make_sc_kernel(e, h, nc, ns, chunk)
    out = sck(p_tab, q_tab, inpk, w2rep)
    o3 = out.reshape(-1, 3, chunk)
    w = lax.bitcast_convert_type(o3[:, 0, :].reshape(e), jnp.float32)
    fs = o3[:, 1, :].reshape(e)
    fd = o3[:, 2, :].reshape(e)
    return w, fs, fd


# speedup vs baseline: 7.7316x; 2.4503x over previous
import functools

import numpy as np
import jax
import jax.numpy as jnp
from jax import lax
from jax.experimental import pallas as pl
from jax.experimental.pallas import tpu as pltpu
from jax.experimental.pallas import tpu_sc as plsc

_LANES = 16
_SCHUNK = 400
_GBATCH = 80

_LO = float(np.log(np.float32(0.01)) - np.log1p(np.float32(-0.01)))
_HI = float(np.log(np.float32(0.99)) - np.log1p(np.float32(-0.99)))


def _tc_tables_body(emb, w1a, w1b, b1r, p_out, q_out):
    x = emb[...]
    p_out[...] = jnp.dot(x, w1a[...], preferred_element_type=jnp.float32)
    q_out[...] = jnp.dot(x, w1b[...], preferred_element_type=jnp.float32) + b1r[...]


def _make_tables(node_emb, W1, b1):
    n, d = node_emb.shape
    h = W1.shape[1]
    w1a = W1[:d]
    w1b = W1[d:]
    b1r = b1.reshape(1, h)
    bm = 1000 if n % 1000 == 0 else n
    grid = n // bm
    return pl.pallas_call(
        _tc_tables_body,
        grid=(grid,),
        in_specs=[
            pl.BlockSpec((bm, d), lambda i: (i, 0)),
            pl.BlockSpec((d, h), lambda i: (0, 0)),
            pl.BlockSpec((d, h), lambda i: (0, 0)),
            pl.BlockSpec((1, h), lambda i: (0, 0)),
        ],
        out_specs=[
            pl.BlockSpec((bm, h), lambda i: (i, 0)),
            pl.BlockSpec((bm, h), lambda i: (i, 0)),
        ],
        out_shape=[
            jax.ShapeDtypeStruct((n, h), jnp.float32),
            jax.ShapeDtypeStruct((n, h), jnp.float32),
        ],
    )(node_emb, w1a, w1b, b1r)


@functools.cache
def _make_sc_kernel(e, h, nc, ns, chunk):
    nw = nc * ns
    per_w = e // nw
    nchunks = per_w // chunk
    ngroups = chunk // _LANES
    nbatch = chunk // _GBATCH
    npairs = (nchunks - 1) // 2
    ilen = 4 * chunk
    olen = 3 * chunk
    mesh = plsc.VectorSubcoreMesh(core_axis_name="c", subcore_axis_name="s")

    @functools.partial(
        pl.kernel,
        out_type=jax.ShapeDtypeStruct((3 * e,), jnp.int32),
        mesh=mesh,
        compiler_params=pltpu.CompilerParams(
            needs_layout_passes=False, use_tc_tiling_on_sc=False),
        scratch_types=[
            pltpu.VMEM((ilen,), jnp.int32),
            pltpu.VMEM((ilen,), jnp.int32),
            pltpu.VMEM((chunk, h), jnp.float32),
            pltpu.VMEM((chunk, h), jnp.float32),
            pltpu.VMEM((chunk, h), jnp.float32),
            pltpu.VMEM((chunk, h), jnp.float32),
            pltpu.VMEM((olen,), jnp.int32),
            pltpu.VMEM((olen,), jnp.int32),
            pltpu.VMEM((h * _LANES,), jnp.float32),
            pltpu.SemaphoreType.DMA,
            pltpu.SemaphoreType.DMA,
            pltpu.SemaphoreType.DMA,
            pltpu.SemaphoreType.DMA,
            pltpu.SemaphoreType.DMA,
            pltpu.SemaphoreType.DMA,
            pltpu.SemaphoreType.DMA,
            pltpu.SemaphoreType.DMA,
        ],
    )
    def sc_kernel(p_hbm, q_hbm, in_hbm, w2_hbm, out_hbm,
                  inb0, inb1, pb0, pb1, qb0, qb1, ob0, ob1, w2v,
                  si0, si1, sp0, sp1, sq0, sq1, so0, so1):
        wid = lax.axis_index("s") * nc + lax.axis_index("c")
        cbase = wid * nchunks
        pltpu.sync_copy(w2_hbm, w2v)
        w2list = [w2v[pl.ds(j * _LANES, _LANES)] for j in range(h)]
        lane_iota = lax.iota(jnp.int32, _LANES)
        lane_row = lane_iota * h

        inbs = (inb0, inb1)
        pbs = (pb0, pb1)
        qbs = (qb0, qb1)
        obs = (ob0, ob1)
        sis = (si0, si1)
        sps = (sp0, sp1)
        sqs = (sq0, sq1)
        sos = (so0, so1)

        def in_desc(g, par):
            return pltpu.make_async_copy(
                in_hbm.at[pl.ds(g * ilen, ilen)], inbs[par], sis[par])

        def start_gathers(par):
            inb = inbs[par]
            for k in range(nbatch):
                pltpu.async_copy(
                    p_hbm.at[inb.at[pl.ds(k * _GBATCH, _GBATCH)]],
                    pbs[par].at[pl.ds(k * _GBATCH, _GBATCH)], sps[par])
                pltpu.async_copy(
                    q_hbm.at[inb.at[pl.ds(chunk + k * _GBATCH, _GBATCH)]],
                    qbs[par].at[pl.ds(k * _GBATCH, _GBATCH)], sqs[par])

        def drain_gathers(par):
            pltpu.make_async_copy(
                p_hbm.at[pl.ds(0, chunk)], pbs[par], sps[par]).wait()
            pltpu.make_async_copy(
                q_hbm.at[pl.ds(0, chunk)], qbs[par], sqs[par]).wait()

        def out_desc(g, par):
            return pltpu.make_async_copy(
                obs[par], out_hbm.at[pl.ds(g * olen, olen)], sos[par])

        def compute(par):
            inb, pb, qb, ob = inbs[par], pbs[par], qbs[par], obs[par]

            def group(gi, carry):
                r0 = gi * _LANES
                rowbase = lane_row + r0 * h
                lpj = lane_iota
                acc = jnp.zeros((_LANES,), jnp.float32)
                for j in range(h):
                    dcol = lax.bitwise_and(lpj, h - 1)
                    idx = rowbase + dcol
                    pc = plsc.load_gather(pb, [jnp.zeros((_LANES,), jnp.int32), idx])
                    qc = plsc.load_gather(qb, [jnp.zeros((_LANES,), jnp.int32), idx])
                    hv = jnp.maximum(pc + qc, 0.0)
                    acc = acc + hv * w2list[j]
                    if j + 1 < h:
                        lpj = lpj + 1
                gn16 = plsc.bitcast(inb[pl.ds(2 * chunk + r0, _LANES)], jnp.float32)
                lg16 = plsc.bitcast(inb[pl.ds(3 * chunk + r0, _LANES)], jnp.float32)
                g16 = acc + gn16
                lp = jnp.minimum(jnp.maximum(g16, _LO), _HI)
                z = (lp + lg16) / 0.9
                wv = 1.0 / (1.0 + jnp.exp(-z))
                keep = wv != 0.0
                s16 = inb[pl.ds(r0, _LANES)]
                d16 = inb[pl.ds(chunk + r0, _LANES)]
                ob[pl.ds(r0, _LANES)] = plsc.bitcast(wv, jnp.int32)
                ob[pl.ds(chunk + r0, _LANES)] = jnp.where(keep, s16, -1)
                ob[pl.ds(2 * chunk + r0, _LANES)] = jnp.where(keep, d16, -1)
                return carry

            lax.fori_loop(0, ngroups, group, 0)

        def sub(c, par, do_next_gather, do_in_guard, store_wait_dynamic):
            g = cbase + c
            drain_gathers(par)
            if do_next_gather:
                in_desc(g + 1, par ^ 1).wait()
                start_gathers(par ^ 1)
            if store_wait_dynamic:
                @pl.when(c >= 2)
                def _():
                    out_desc(g - 2, par).wait()
            else:
                out_desc(g - 2, par).wait()
            compute(par)
            out_desc(g, par).start()
            if do_in_guard == "always":
                in_desc(g + 2, par).start()
            elif do_in_guard == "guard":
                @pl.when(c + 2 <= nchunks - 1)
                def _():
                    in_desc(g + 2, par).start()

        in_desc(cbase, 0).start()
        in_desc(cbase + 1, 1).start()
        in_desc(cbase, 0).wait()
        start_gathers(0)

        def pair(i, carry):
            c0 = 2 * i
            sub(c0, 0, True, "always", True)
            sub(c0 + 1, 1, True, "guard", True)
            return carry

        lax.fori_loop(0, npairs, pair, 0)
        sub(nchunks - 1, 0, False, "none", False)
        out_desc(cbase + nchunks - 2, 1).wait()
        out_desc(cbase + nchunks - 1, 0).wait()

    return sc_kernel


def kernel(node_emb, edge_index, W1, b1, W2, b2):
    e = edge_index.shape[1]
    h = W1.shape[1]
    chunk = _SCHUNK

    p_tab, q_tab = _make_tables(node_emb, W1, b1)
    src = edge_index[0]
    dst = edge_index[1]

    u = jax.random.uniform(jax.random.key(1), (e, 1), dtype=jnp.float32)[:, 0]
    bias = 0.0001
    eps = (bias - (1.0 - bias)) * u + (1.0 - bias)
    gn = jnp.log(eps) - jnp.log(1.0 - eps) + b2[0]
    u2 = jax.random.uniform(jax.random.key(2), (e,), minval=1e-7,
                            maxval=1.0 - 1e-7, dtype=jnp.float32)
    lg = jnp.log(u2) - jnp.log(1.0 - u2)

    jj = (jnp.arange(h)[:, None] + jnp.arange(_LANES)[None, :]) % h
    w2rot = W2[:, 0][jj].reshape(-1)

    gnb = lax.bitcast_convert_type(gn, jnp.int32)
    lgb = lax.bitcast_convert_type(lg, jnp.int32)
    inpk = jnp.stack(
        [src.reshape(-1, chunk), dst.reshape(-1, chunk),
         gnb.reshape(-1, chunk), lgb.reshape(-1, chunk)], axis=1).reshape(-1)

    try:
        info = plsc.get_sparse_core_info()
        nc, ns = info.num_cores, info.num_subcores
    except Exception:
        nc, ns = 2, 16

    sck = _make_sc_kernel(e, h, nc, ns, chunk)
    out = sck(p_tab, q_tab, inpk, w2rot)
    o3 = out.reshape(-1, 3, chunk)
    w = lax.bitcast_convert_type(o3[:, 0, :].reshape(e), jnp.float32)
    fs = o3[:, 1, :].reshape(e)
    fd = o3[:, 2, :].reshape(e)
    return w, fs, fd
